# speedup vs baseline: 1.7398x; 1.7398x over previous
import functools

import jax
import jax.numpy as jnp
from jax.experimental import pallas as pl
from jax.experimental.pallas import tpu as pltpu

NUM_NODES = 100000
E = 320000
D = 128
D_EDGE = 16
N_TYPES = 8


def _edge_h_body(msg_ref, ef_ref, wm_ref, we_ref, out_ref):
    out_ref[...] = (
        jnp.dot(msg_ref[...], wm_ref[...], preferred_element_type=jnp.float32)
        + jnp.dot(ef_ref[...], we_ref[...], preferred_element_type=jnp.float32)
    )


def _edge_h(msg, ef, W_msg, W_ef):
    blk = 1600
    grid = (E // blk,)
    return pl.pallas_call(
        _edge_h_body,
        grid=grid,
        in_specs=[
            pl.BlockSpec((blk, D_EDGE), lambda i: (i, 0)),
            pl.BlockSpec((blk, D_EDGE), lambda i: (i, 0)),
            pl.BlockSpec((D_EDGE, D), lambda i: (0, 0)),
            pl.BlockSpec((D_EDGE, D), lambda i: (0, 0)),
        ],
        out_specs=pl.BlockSpec((blk, D), lambda i: (i, 0)),
        out_shape=jax.ShapeDtypeStruct((E, D), jnp.float32),
    )(msg, ef, W_msg, W_ef)


def _node_enc_body(wxs_ref, wxd_ref, wehs_ref, wehd_ref, hd_ref, wenc_ref,
                   a_ref, b_ref):
    S = jax.nn.relu(
        jnp.dot(wxs_ref[...], wenc_ref[...], preferred_element_type=jnp.float32))
    T = jax.nn.relu(
        jnp.dot(wxd_ref[...], wenc_ref[...], preferred_element_type=jnp.float32))
    hd = hd_ref[...]
    WH = jnp.where(hd > 0.5, T + wehd_ref[...], S + wehs_ref[...])
    a_ref[...] = jnp.concatenate([S, WH], axis=1)
    b_ref[...] = jnp.concatenate([T, WH], axis=1)


def _node_enc(wx_src, wx_dst, WEHs, WEHd, has_dst, W_enc, n_rows):
    blk = 800
    grid = (n_rows // blk,)
    return pl.pallas_call(
        _node_enc_body,
        grid=grid,
        in_specs=[
            pl.BlockSpec((blk, D), lambda i: (i, 0)),
            pl.BlockSpec((blk, D), lambda i: (i, 0)),
            pl.BlockSpec((blk, D), lambda i: (i, 0)),
            pl.BlockSpec((blk, D), lambda i: (i, 0)),
            pl.BlockSpec((blk, 1), lambda i: (i, 0)),
            pl.BlockSpec((D, D), lambda i: (0, 0)),
        ],
        out_specs=[
            pl.BlockSpec((blk, 2 * D), lambda i: (i, 0)),
            pl.BlockSpec((blk, 2 * D), lambda i: (i, 0)),
        ],
        out_shape=[
            jax.ShapeDtypeStruct((n_rows, 2 * D), jnp.float32),
            jax.ShapeDtypeStruct((n_rows, 2 * D), jnp.float32),
        ],
    )(wx_src, wx_dst, WEHs, WEHd, has_dst, W_enc)


def _final_body(gs_ref, gd_ref, msg_ref, ef_ref, et_ref, wm_ref, we_ref,
                wdec_ref, tb_ref, out_ref, acc_ref):
    i = pl.program_id(0)

    @pl.when(i == 0)
    def _():
        acc_ref[0] = 0.0
        acc_ref[1] = 0.0

    EH = (jnp.dot(msg_ref[...], wm_ref[...], preferred_element_type=jnp.float32)
          + jnp.dot(ef_ref[...], we_ref[...], preferred_element_type=jnp.float32))
    gs = gs_ref[...]
    gd = gd_ref[...]
    h_src = gs[:, :D] + EH
    h_dst = gd[:, :D] + EH
    hdw = jnp.dot(h_dst, wdec_ref[...], preferred_element_type=jnp.float32)
    score = jnp.sum(h_src * hdw, axis=1)
    et = et_ref[0, 0, :]
    bias = jnp.zeros_like(score)
    for k in range(N_TYPES):
        bias += jnp.where(et == k, tb_ref[k], 0.0)
    score = score + bias
    sp = jnp.maximum(-score, 0.0) + jnp.log1p(jnp.exp(-jnp.abs(score)))
    ds = h_src - gs[:, D:]
    dd = h_dst - gd[:, D:]
    acc_ref[0] += jnp.sum(sp)
    acc_ref[1] += jnp.sum(ds * ds) + jnp.sum(dd * dd)

    @pl.when(i == pl.num_programs(0) - 1)
    def _():
        out_ref[0] = acc_ref[0] / E + 0.1 * (acc_ref[1] / (E * D))


def _final(Gs, Gd, msg, ef, edge_type, W_msg, W_ef, W_dec, type_bias):
    blk = 512
    grid = (E // blk,)
    et3 = edge_type.astype(jnp.int32).reshape(E // blk, 1, blk)
    return pl.pallas_call(
        _final_body,
        grid=grid,
        in_specs=[
            pl.BlockSpec((blk, 2 * D), lambda i: (i, 0)),
            pl.BlockSpec((blk, 2 * D), lambda i: (i, 0)),
            pl.BlockSpec((blk, D_EDGE), lambda i: (i, 0)),
            pl.BlockSpec((blk, D_EDGE), lambda i: (i, 0)),
            pl.BlockSpec((1, 1, blk), lambda i: (i, 0, 0)),
            pl.BlockSpec((D_EDGE, D), lambda i: (0, 0)),
            pl.BlockSpec((D_EDGE, D), lambda i: (0, 0)),
            pl.BlockSpec((D, D), lambda i: (0, 0)),
            pl.BlockSpec(memory_space=pltpu.SMEM),
        ],
        out_specs=pl.BlockSpec(memory_space=pltpu.SMEM),
        out_shape=jax.ShapeDtypeStruct((1,), jnp.float32),
        scratch_shapes=[pltpu.SMEM((2,), jnp.float32)],
        compiler_params=pltpu.CompilerParams(
            dimension_semantics=("arbitrary",)),
    )(Gs, Gd, msg, ef, et3, W_msg, W_ef, W_dec, type_bias)


def kernel(x_src, x_dst, msg, edge_feats, W_enc, W_msg, W_ef, W_dec, type_bias,
           last_h_storage, src, dst, t, edge_type):
    src = src.astype(jnp.int32)
    dst = dst.astype(jnp.int32)
    e_ids = jnp.arange(E, dtype=jnp.int32)

    lastsrc = jnp.full((NUM_NODES,), -1, jnp.int32).at[src].max(e_ids)
    lastdst = jnp.full((NUM_NODES,), -1, jnp.int32).at[dst].max(e_ids)
    lsc = jnp.maximum(lastsrc, 0)
    ldc = jnp.maximum(lastdst, 0)
    has_dst = (lastdst >= 0).astype(jnp.float32).reshape(NUM_NODES, 1)

    EH = _edge_h(msg, edge_feats, W_msg, W_ef)

    wx_src = x_src[lsc]
    wx_dst = x_dst[ldc]
    WEHs = EH[lsc]
    WEHd = EH[ldc]

    A, B = _node_enc(wx_src, wx_dst, WEHs, WEHd, has_dst, W_enc, NUM_NODES)

    Gs = A[src]
    Gd = B[dst]

    return _final(Gs, Gd, msg, edge_feats, edge_type, W_msg, W_ef, W_dec,
                  type_bias)


# speedup vs baseline: 2.3093x; 1.3273x over previous
import functools

import jax
import jax.numpy as jnp
from jax import lax
from jax.experimental import pallas as pl
from jax.experimental.pallas import tpu as pltpu
from jax.experimental.pallas import tpu_sc as plsc

NUM_NODES = 100000
E = 320000
D = 128
D_EDGE = 16
N_TYPES = 8

NTILES = 32
NPAD = 100096
EPW = E // NTILES
RPW = NPAD // NTILES


def _mesh():
    return plsc.VectorSubcoreMesh(core_axis_name="c", subcore_axis_name="s")


def _wid():
    return lax.axis_index("s") * 2 + lax.axis_index("c")


def _sc_winner_tables(src, dst):
    grp = EPW // 16

    @functools.partial(
        pl.kernel,
        mesh=_mesh(),
        compiler_params=pltpu.CompilerParams(needs_layout_passes=False),
        out_type=[
            jax.ShapeDtypeStruct((NTILES, NPAD), jnp.int32),
            jax.ShapeDtypeStruct((NTILES, NPAD), jnp.int32),
        ],
        scratch_types=[
            pltpu.VMEM((NPAD,), jnp.int32),
            pltpu.VMEM((EPW,), jnp.int32),
        ],
    )
    def k(src_hbm, dst_hbm, ls_hbm, ld_hbm, tbl, chunk):
        wid = _wid()
        lane = lax.iota(jnp.int32, 16)

        def one_direction(ids_hbm, out_hbm):
            pltpu.sync_copy(ids_hbm.at[pl.ds(wid * EPW, EPW)], chunk)

            def init_body(i, _):
                tbl[pl.ds(i * 16, 16)] = jnp.full((16,), jnp.int32(-1),
                                                  jnp.int32)
                return 0

            lax.fori_loop(0, NPAD // 16, init_body, 0)

            def scat_body(g, _):
                node = chunk[pl.ds(g * 16, 16)]
                ev = wid * EPW + g * 16 + lane
                for j in range(16):
                    plsc.store_scatter(tbl, [node], ev, mask=lane == j)
                return 0

            lax.fori_loop(0, grp, scat_body, 0)
            pltpu.sync_copy(tbl, out_hbm.at[wid])

        one_direction(src_hbm, ls_hbm)
        one_direction(dst_hbm, ld_hbm)

    return k(src, dst)


def _merge_body(ls_ref, ld_ref, lsc_ref, ldc_ref, hd_ref):
    ms = jnp.max(ls_ref[...], axis=0)
    md = jnp.max(ld_ref[...], axis=0)
    lsc_ref[...] = jnp.maximum(ms, 0)
    ldc_ref[...] = jnp.maximum(md, 0)
    hd_ref[...] = (md >= 0).astype(jnp.float32)


def _merge_tc(Ls, Ld):
    return pl.pallas_call(
        _merge_body,
        out_shape=[
            jax.ShapeDtypeStruct((NPAD,), jnp.int32),
            jax.ShapeDtypeStruct((NPAD,), jnp.int32),
            jax.ShapeDtypeStruct((NPAD,), jnp.float32),
        ],
    )(Ls, Ld)


def _sc_node_gathers(x_src, x_dst, EH, lsc, ldc):
    C = 184
    nchunk = RPW // C

    @functools.partial(
        pl.kernel,
        mesh=_mesh(),
        compiler_params=pltpu.CompilerParams(needs_layout_passes=False),
        out_type=[jax.ShapeDtypeStruct((NPAD, D), jnp.float32)] * 4,
        scratch_types=[
            pltpu.VMEM((RPW,), jnp.int32),
            pltpu.VMEM((RPW,), jnp.int32),
            pltpu.VMEM((C, D), jnp.float32),
            pltpu.SemaphoreType.DMA,
        ],
    )
    def k(xs_hbm, xd_hbm, eh_hbm, lsc_hbm, ldc_hbm,
          wxs_hbm, wxd_hbm, wehs_hbm, wehd_hbm, idx_s, idx_d, buf, sem):
        wid = _wid()
        base = wid * RPW
        pltpu.sync_copy(lsc_hbm.at[pl.ds(base, RPW)], idx_s)
        pltpu.sync_copy(ldc_hbm.at[pl.ds(base, RPW)], idx_d)

        def job(tab_hbm, idx, out_hbm):
            def body(j, _):
                pltpu.async_copy(
                    tab_hbm.at[idx.at[pl.ds(j * C, C)]], buf, sem).wait()
                pltpu.sync_copy(buf, out_hbm.at[pl.ds(base + j * C, C)])
                return 0

            lax.fori_loop(0, nchunk, body, 0)

        job(xs_hbm, idx_s, wxs_hbm)
        job(eh_hbm, idx_s, wehs_hbm)
        job(xd_hbm, idx_d, wxd_hbm)
        job(eh_hbm, idx_d, wehd_hbm)

    return k(x_src, x_dst, EH, lsc, ldc)


def _sc_edge_gathers(A, B, src, dst):
    C = 200
    nchunk = EPW // C

    @functools.partial(
        pl.kernel,
        mesh=_mesh(),
        compiler_params=pltpu.CompilerParams(needs_layout_passes=False),
        out_type=[jax.ShapeDtypeStruct((E, 2 * D), jnp.float32)] * 2,
        scratch_types=[
            pltpu.VMEM((EPW,), jnp.int32),
            pltpu.VMEM((C, 2 * D), jnp.float32),
            pltpu.SemaphoreType.DMA,
        ],
    )
    def k(a_hbm, b_hbm, src_hbm, dst_hbm, gs_hbm, gd_hbm, idx, buf, sem):
        wid = _wid()
        base = wid * EPW

        def job(tab_hbm, ids_hbm, out_hbm):
            pltpu.sync_copy(ids_hbm.at[pl.ds(base, EPW)], idx)

            def body(j, _):
                pltpu.async_copy(
                    tab_hbm.at[idx.at[pl.ds(j * C, C)]], buf, sem).wait()
                pltpu.sync_copy(buf, out_hbm.at[pl.ds(base + j * C, C)])
                return 0

            lax.fori_loop(0, nchunk, body, 0)

        job(a_hbm, src_hbm, gs_hbm)
        job(b_hbm, dst_hbm, gd_hbm)

    return k(A, B, src, dst)


def _edge_h_body(msg_ref, ef_ref, wm_ref, we_ref, out_ref):
    out_ref[...] = (
        jnp.dot(msg_ref[...], wm_ref[...], preferred_element_type=jnp.float32)
        + jnp.dot(ef_ref[...], we_ref[...], preferred_element_type=jnp.float32)
    )


def _edge_h(msg, ef, W_msg, W_ef):
    blk = 1600
    grid = (E // blk,)
    return pl.pallas_call(
        _edge_h_body,
        grid=grid,
        in_specs=[
            pl.BlockSpec((blk, D_EDGE), lambda i: (i, 0)),
            pl.BlockSpec((blk, D_EDGE), lambda i: (i, 0)),
            pl.BlockSpec((D_EDGE, D), lambda i: (0, 0)),
            pl.BlockSpec((D_EDGE, D), lambda i: (0, 0)),
        ],
        out_specs=pl.BlockSpec((blk, D), lambda i: (i, 0)),
        out_shape=jax.ShapeDtypeStruct((E, D), jnp.float32),
    )(msg, ef, W_msg, W_ef)


def _node_enc_body(wxs_ref, wxd_ref, wehs_ref, wehd_ref, hd_ref, wenc_ref,
                   a_ref, b_ref):
    S = jax.nn.relu(
        jnp.dot(wxs_ref[...], wenc_ref[...], preferred_element_type=jnp.float32))
    T = jax.nn.relu(
        jnp.dot(wxd_ref[...], wenc_ref[...], preferred_element_type=jnp.float32))
    hd = hd_ref[...]
    WH = jnp.where(hd > 0.5, T + wehd_ref[...], S + wehs_ref[...])
    a_ref[...] = jnp.concatenate([S, WH], axis=1)
    b_ref[...] = jnp.concatenate([T, WH], axis=1)


def _node_enc(wx_src, wx_dst, WEHs, WEHd, has_dst, W_enc, n_rows):
    blk = 3128
    grid = (n_rows // blk,)
    return pl.pallas_call(
        _node_enc_body,
        grid=grid,
        in_specs=[
            pl.BlockSpec((blk, D), lambda i: (i, 0)),
            pl.BlockSpec((blk, D), lambda i: (i, 0)),
            pl.BlockSpec((blk, D), lambda i: (i, 0)),
            pl.BlockSpec((blk, D), lambda i: (i, 0)),
            pl.BlockSpec((blk, 1), lambda i: (i, 0)),
            pl.BlockSpec((D, D), lambda i: (0, 0)),
        ],
        out_specs=[
            pl.BlockSpec((blk, 2 * D), lambda i: (i, 0)),
            pl.BlockSpec((blk, 2 * D), lambda i: (i, 0)),
        ],
        out_shape=[
            jax.ShapeDtypeStruct((n_rows, 2 * D), jnp.float32),
            jax.ShapeDtypeStruct((n_rows, 2 * D), jnp.float32),
        ],
    )(wx_src, wx_dst, WEHs, WEHd, has_dst, W_enc)


def _final_body(gs_ref, gd_ref, msg_ref, ef_ref, et_ref, wm_ref, we_ref,
                wdec_ref, tb_ref, out_ref, acc_ref):
    i = pl.program_id(0)

    @pl.when(i == 0)
    def _():
        acc_ref[0] = 0.0
        acc_ref[1] = 0.0

    EH = (jnp.dot(msg_ref[...], wm_ref[...], preferred_element_type=jnp.float32)
          + jnp.dot(ef_ref[...], we_ref[...], preferred_element_type=jnp.float32))
    gs = gs_ref[...]
    gd = gd_ref[...]
    h_src = gs[:, :D] + EH
    h_dst = gd[:, :D] + EH
    hdw = jnp.dot(h_dst, wdec_ref[...], preferred_element_type=jnp.float32)
    score = jnp.sum(h_src * hdw, axis=1)
    et = et_ref[0, 0, :]
    bias = jnp.zeros_like(score)
    for k in range(N_TYPES):
        bias += jnp.where(et == k, tb_ref[k], 0.0)
    score = score + bias
    sp = jnp.maximum(-score, 0.0) + jnp.log1p(jnp.exp(-jnp.abs(score)))
    ds = h_src - gs[:, D:]
    dd = h_dst - gd[:, D:]
    acc_ref[0] += jnp.sum(sp)
    acc_ref[1] += jnp.sum(ds * ds) + jnp.sum(dd * dd)

    @pl.when(i == pl.num_programs(0) - 1)
    def _():
        out_ref[0] = acc_ref[0] / E + 0.1 * (acc_ref[1] / (E * D))


def _final(Gs, Gd, msg, ef, edge_type, W_msg, W_ef, W_dec, type_bias):
    blk = 512
    grid = (E // blk,)
    et3 = edge_type.astype(jnp.int32).reshape(E // blk, 1, blk)
    return pl.pallas_call(
        _final_body,
        grid=grid,
        in_specs=[
            pl.BlockSpec((blk, 2 * D), lambda i: (i, 0)),
            pl.BlockSpec((blk, 2 * D), lambda i: (i, 0)),
            pl.BlockSpec((blk, D_EDGE), lambda i: (i, 0)),
            pl.BlockSpec((blk, D_EDGE), lambda i: (i, 0)),
            pl.BlockSpec((1, 1, blk), lambda i: (i, 0, 0)),
            pl.BlockSpec((D_EDGE, D), lambda i: (0, 0)),
            pl.BlockSpec((D_EDGE, D), lambda i: (0, 0)),
            pl.BlockSpec((D, D), lambda i: (0, 0)),
            pl.BlockSpec(memory_space=pltpu.SMEM),
        ],
        out_specs=pl.BlockSpec(memory_space=pltpu.SMEM),
        out_shape=jax.ShapeDtypeStruct((1,), jnp.float32),
        scratch_shapes=[pltpu.SMEM((2,), jnp.float32)],
        compiler_params=pltpu.CompilerParams(
            dimension_semantics=("arbitrary",)),
    )(Gs, Gd, msg, ef, et3, W_msg, W_ef, W_dec, type_bias)


def kernel(x_src, x_dst, msg, edge_feats, W_enc, W_msg, W_ef, W_dec, type_bias,
           last_h_storage, src, dst, t, edge_type):
    src = src.astype(jnp.int32)
    dst = dst.astype(jnp.int32)

    Ls, Ld = _sc_winner_tables(src, dst)
    lsc, ldc, hd = _merge_tc(Ls, Ld)

    EH = _edge_h(msg, edge_feats, W_msg, W_ef)

    wx_src, wx_dst, WEHs, WEHd = _sc_node_gathers(x_src, x_dst, EH, lsc, ldc)

    A, B = _node_enc(wx_src, wx_dst, WEHs, WEHd, hd.reshape(NPAD, 1), W_enc,
                     NPAD)

    Gs, Gd = _sc_edge_gathers(A, B, src, dst)

    return _final(Gs, Gd, msg, edge_feats, edge_type, W_msg, W_ef, W_dec,
                  type_bias)


# speedup vs baseline: 2.4091x; 1.0432x over previous
import functools

import jax
import jax.numpy as jnp
from jax import lax
from jax.experimental import pallas as pl
from jax.experimental.pallas import tpu as pltpu
from jax.experimental.pallas import tpu_sc as plsc

NUM_NODES = 100000
E = 320000
D = 128
D_EDGE = 16
N_TYPES = 8

NTILES = 32
NPAD = 100096
EPW = E // NTILES
RPW = NPAD // NTILES


def _mesh():
    return plsc.VectorSubcoreMesh(core_axis_name="c", subcore_axis_name="s")


def _wid():
    return lax.axis_index("s") * 2 + lax.axis_index("c")


def _ring_gather_job(tab, idxref, out, base, C, nchunk, buf_a, buf_b,
                     sem_a, sem_b):
    def sg(j, buf, sem):
        pltpu.async_copy(tab.at[idxref.at[pl.ds(j * C, C)]], buf, sem)

    def wg(buf, sem):
        pltpu.make_async_copy(tab.at[pl.ds(0, C)], buf, sem).wait()

    def out_cp(j, buf):
        pltpu.sync_copy(buf, out.at[pl.ds(base + j * C, C)])

    sg(0, buf_a, sem_a)
    sg(1, buf_b, sem_b)
    npair = (nchunk - 2) // 2 if nchunk % 2 == 0 else (nchunk - 3) // 2

    def body(j2, _):
        j = 2 * j2
        wg(buf_a, sem_a)
        out_cp(j, buf_a)
        sg(j + 2, buf_a, sem_a)
        wg(buf_b, sem_b)
        out_cp(j + 1, buf_b)
        sg(j + 3, buf_b, sem_b)
        return 0

    lax.fori_loop(0, npair, body, 0)
    if nchunk % 2 == 0:
        wg(buf_a, sem_a)
        out_cp(nchunk - 2, buf_a)
        wg(buf_b, sem_b)
        out_cp(nchunk - 1, buf_b)
    else:
        wg(buf_a, sem_a)
        out_cp(nchunk - 3, buf_a)
        sg(nchunk - 1, buf_a, sem_a)
        wg(buf_b, sem_b)
        out_cp(nchunk - 2, buf_b)
        wg(buf_a, sem_a)
        out_cp(nchunk - 1, buf_a)


def _sc_winner_tables(src, dst):
    grp = EPW // 16

    @functools.partial(
        pl.kernel,
        mesh=_mesh(),
        compiler_params=pltpu.CompilerParams(needs_layout_passes=False),
        out_type=[
            jax.ShapeDtypeStruct((NTILES, NPAD), jnp.int32),
            jax.ShapeDtypeStruct((NTILES, NPAD), jnp.int32),
        ],
        scratch_types=[
            pltpu.VMEM((NPAD,), jnp.int32),
            pltpu.VMEM((EPW,), jnp.int32),
        ],
    )
    def k(src_hbm, dst_hbm, ls_hbm, ld_hbm, tbl, chunk):
        wid = _wid()
        lane = lax.iota(jnp.int32, 16)

        def one_direction(ids_hbm, out_hbm):
            pltpu.sync_copy(ids_hbm.at[pl.ds(wid * EPW, EPW)], chunk)

            def init_body(i, _):
                tbl[pl.ds(i * 16, 16)] = jnp.full((16,), jnp.int32(-1),
                                                  jnp.int32)
                return 0

            lax.fori_loop(0, NPAD // 16, init_body, 0)

            def scat_body(g, _):
                node = chunk[pl.ds(g * 16, 16)]
                ev = wid * EPW + g * 16 + lane
                for j in range(16):
                    plsc.store_scatter(tbl, [node], ev, mask=lane == j)
                return 0

            lax.fori_loop(0, grp, scat_body, 0)
            pltpu.sync_copy(tbl, out_hbm.at[wid])

        one_direction(src_hbm, ls_hbm)
        one_direction(dst_hbm, ld_hbm)

    return k(src, dst)


def _merge_body(ls_ref, ld_ref, lsc_ref, ldc_ref, hd_ref):
    ms = jnp.max(ls_ref[...], axis=0)
    md = jnp.max(ld_ref[...], axis=0)
    lsc_ref[...] = jnp.maximum(ms, 0)
    ldc_ref[...] = jnp.maximum(md, 0)
    hd_ref[...] = (md >= 0).astype(jnp.float32)


def _merge_tc(Ls, Ld):
    return pl.pallas_call(
        _merge_body,
        out_shape=[
            jax.ShapeDtypeStruct((NPAD,), jnp.int32),
            jax.ShapeDtypeStruct((NPAD,), jnp.int32),
            jax.ShapeDtypeStruct((NPAD,), jnp.float32),
        ],
    )(Ls, Ld)


def _sc_node_gathers(x_src, x_dst, EH, lsc, ldc):
    C = 184
    nchunk = RPW // C

    @functools.partial(
        pl.kernel,
        mesh=_mesh(),
        compiler_params=pltpu.CompilerParams(needs_layout_passes=False),
        out_type=[jax.ShapeDtypeStruct((NPAD, D), jnp.float32)] * 4,
        scratch_types=[
            pltpu.VMEM((RPW,), jnp.int32),
            pltpu.VMEM((RPW,), jnp.int32),
            pltpu.VMEM((C, D), jnp.float32),
            pltpu.VMEM((C, D), jnp.float32),
            pltpu.SemaphoreType.DMA,
            pltpu.SemaphoreType.DMA,
        ],
    )
    def k(xs_hbm, xd_hbm, eh_hbm, lsc_hbm, ldc_hbm,
          wxs_hbm, wxd_hbm, wehs_hbm, wehd_hbm, idx_s, idx_d,
          buf_a, buf_b, sem_a, sem_b):
        wid = _wid()
        base = wid * RPW
        pltpu.sync_copy(lsc_hbm.at[pl.ds(base, RPW)], idx_s)
        pltpu.sync_copy(ldc_hbm.at[pl.ds(base, RPW)], idx_d)
        for tab_hbm, idx, out_hbm in (
                (xs_hbm, idx_s, wxs_hbm), (eh_hbm, idx_s, wehs_hbm),
                (xd_hbm, idx_d, wxd_hbm), (eh_hbm, idx_d, wehd_hbm)):
            _ring_gather_job(tab_hbm, idx, out_hbm, base, C, nchunk,
                             buf_a, buf_b, sem_a, sem_b)

    return k(x_src, x_dst, EH, lsc, ldc)


def _sc_edge_gathers(A, B, src, dst):
    C = 200
    nchunk = EPW // C

    @functools.partial(
        pl.kernel,
        mesh=_mesh(),
        compiler_params=pltpu.CompilerParams(needs_layout_passes=False),
        out_type=[jax.ShapeDtypeStruct((E, 2 * D), jnp.float32)] * 2,
        scratch_types=[
            pltpu.VMEM((EPW,), jnp.int32),
            pltpu.VMEM((C, 2 * D), jnp.float32),
            pltpu.VMEM((C, 2 * D), jnp.float32),
            pltpu.SemaphoreType.DMA,
            pltpu.SemaphoreType.DMA,
        ],
    )
    def k(a_hbm, b_hbm, src_hbm, dst_hbm, gs_hbm, gd_hbm, idx,
          buf_a, buf_b, sem_a, sem_b):
        wid = _wid()
        base = wid * EPW

        def job(tab_hbm, ids_hbm, out_hbm):
            pltpu.sync_copy(ids_hbm.at[pl.ds(base, EPW)], idx)
            _ring_gather_job(tab_hbm, idx, out_hbm, base, C, nchunk,
                             buf_a, buf_b, sem_a, sem_b)

        job(a_hbm, src_hbm, gs_hbm)
        job(b_hbm, dst_hbm, gd_hbm)

    return k(A, B, src, dst)


def _edge_h_body(msg_ref, ef_ref, wm_ref, we_ref, out_ref):
    out_ref[...] = (
        jnp.dot(msg_ref[...], wm_ref[...], preferred_element_type=jnp.float32)
        + jnp.dot(ef_ref[...], we_ref[...], preferred_element_type=jnp.float32)
    )


def _edge_h(msg, ef, W_msg, W_ef):
    blk = 1600
    grid = (E // blk,)
    return pl.pallas_call(
        _edge_h_body,
        grid=grid,
        in_specs=[
            pl.BlockSpec((blk, D_EDGE), lambda i: (i, 0)),
            pl.BlockSpec((blk, D_EDGE), lambda i: (i, 0)),
            pl.BlockSpec((D_EDGE, D), lambda i: (0, 0)),
            pl.BlockSpec((D_EDGE, D), lambda i: (0, 0)),
        ],
        out_specs=pl.BlockSpec((blk, D), lambda i: (i, 0)),
        out_shape=jax.ShapeDtypeStruct((E, D), jnp.float32),
    )(msg, ef, W_msg, W_ef)


def _node_enc_body(wxs_ref, wxd_ref, wehs_ref, wehd_ref, hd_ref, wenc_ref,
                   a_ref, b_ref):
    S = jax.nn.relu(
        jnp.dot(wxs_ref[...], wenc_ref[...], preferred_element_type=jnp.float32))
    T = jax.nn.relu(
        jnp.dot(wxd_ref[...], wenc_ref[...], preferred_element_type=jnp.float32))
    hd = hd_ref[...]
    WH = jnp.where(hd > 0.5, T + wehd_ref[...], S + wehs_ref[...])
    a_ref[...] = jnp.concatenate([S, WH], axis=1)
    b_ref[...] = jnp.concatenate([T, WH], axis=1)


def _node_enc(wx_src, wx_dst, WEHs, WEHd, has_dst, W_enc, n_rows):
    blk = 3128
    grid = (n_rows // blk,)
    return pl.pallas_call(
        _node_enc_body,
        grid=grid,
        in_specs=[
            pl.BlockSpec((blk, D), lambda i: (i, 0)),
            pl.BlockSpec((blk, D), lambda i: (i, 0)),
            pl.BlockSpec((blk, D), lambda i: (i, 0)),
            pl.BlockSpec((blk, D), lambda i: (i, 0)),
            pl.BlockSpec((blk, 1), lambda i: (i, 0)),
            pl.BlockSpec((D, D), lambda i: (0, 0)),
        ],
        out_specs=[
            pl.BlockSpec((blk, 2 * D), lambda i: (i, 0)),
            pl.BlockSpec((blk, 2 * D), lambda i: (i, 0)),
        ],
        out_shape=[
            jax.ShapeDtypeStruct((n_rows, 2 * D), jnp.float32),
            jax.ShapeDtypeStruct((n_rows, 2 * D), jnp.float32),
        ],
    )(wx_src, wx_dst, WEHs, WEHd, has_dst, W_enc)


def _final_body(gs_ref, gd_ref, msg_ref, ef_ref, et_ref, wm_ref, we_ref,
                wdec_ref, tb_ref, out_ref, acc_ref):
    i = pl.program_id(0)

    @pl.when(i == 0)
    def _():
        acc_ref[0] = 0.0
        acc_ref[1] = 0.0

    EH = (jnp.dot(msg_ref[...], wm_ref[...], preferred_element_type=jnp.float32)
          + jnp.dot(ef_ref[...], we_ref[...], preferred_element_type=jnp.float32))
    gs = gs_ref[...]
    gd = gd_ref[...]
    h_src = gs[:, :D] + EH
    h_dst = gd[:, :D] + EH
    hdw = jnp.dot(h_dst, wdec_ref[...], preferred_element_type=jnp.float32)
    score = jnp.sum(h_src * hdw, axis=1)
    et = et_ref[0, 0, :]
    bias = jnp.zeros_like(score)
    for k in range(N_TYPES):
        bias += jnp.where(et == k, tb_ref[k], 0.0)
    score = score + bias
    sp = jnp.maximum(-score, 0.0) + jnp.log1p(jnp.exp(-jnp.abs(score)))
    ds = h_src - gs[:, D:]
    dd = h_dst - gd[:, D:]
    acc_ref[0] += jnp.sum(sp)
    acc_ref[1] += jnp.sum(ds * ds) + jnp.sum(dd * dd)

    @pl.when(i == pl.num_programs(0) - 1)
    def _():
        out_ref[0] = acc_ref[0] / E + 0.1 * (acc_ref[1] / (E * D))


def _final(Gs, Gd, msg, ef, edge_type, W_msg, W_ef, W_dec, type_bias):
    blk = 512
    grid = (E // blk,)
    et3 = edge_type.astype(jnp.int32).reshape(E // blk, 1, blk)
    return pl.pallas_call(
        _final_body,
        grid=grid,
        in_specs=[
            pl.BlockSpec((blk, 2 * D), lambda i: (i, 0)),
            pl.BlockSpec((blk, 2 * D), lambda i: (i, 0)),
            pl.BlockSpec((blk, D_EDGE), lambda i: (i, 0)),
            pl.BlockSpec((blk, D_EDGE), lambda i: (i, 0)),
            pl.BlockSpec((1, 1, blk), lambda i: (i, 0, 0)),
            pl.BlockSpec((D_EDGE, D), lambda i: (0, 0)),
            pl.BlockSpec((D_EDGE, D), lambda i: (0, 0)),
            pl.BlockSpec((D, D), lambda i: (0, 0)),
            pl.BlockSpec(memory_space=pltpu.SMEM),
        ],
        out_specs=pl.BlockSpec(memory_space=pltpu.SMEM),
        out_shape=jax.ShapeDtypeStruct((1,), jnp.float32),
        scratch_shapes=[pltpu.SMEM((2,), jnp.float32)],
        compiler_params=pltpu.CompilerParams(
            dimension_semantics=("arbitrary",)),
    )(Gs, Gd, msg, ef, et3, W_msg, W_ef, W_dec, type_bias)


def kernel(x_src, x_dst, msg, edge_feats, W_enc, W_msg, W_ef, W_dec, type_bias,
           last_h_storage, src, dst, t, edge_type):
    src = src.astype(jnp.int32)
    dst = dst.astype(jnp.int32)

    Ls, Ld = _sc_winner_tables(src, dst)
    lsc, ldc, hd = _merge_tc(Ls, Ld)

    EH = _edge_h(msg, edge_feats, W_msg, W_ef)

    wx_src, wx_dst, WEHs, WEHd = _sc_node_gathers(x_src, x_dst, EH, lsc, ldc)

    A, B = _node_enc(wx_src, wx_dst, WEHs, WEHd, hd.reshape(NPAD, 1), W_enc,
                     NPAD)

    Gs, Gd = _sc_edge_gathers(A, B, src, dst)

    return _final(Gs, Gd, msg, edge_feats, edge_type, W_msg, W_ef, W_dec,
                  type_bias)
